# Initial kernel scaffold; baseline (speedup 1.0000x reference)
import jax
import jax.numpy as jnp
from jax.experimental import pallas as pl


def kernel(x, batch):
    raise NotImplementedError("write your pallas kernel here")


# speedup vs baseline: 1.0821x; 1.0821x over previous
import functools

import jax
import jax.numpy as jnp
from jax import lax
from jax.experimental import pallas as pl
from jax.experimental.pallas import tpu as pltpu
from jax.experimental.pallas import tpu_sc as plsc

N = 8192
D = 128
BR = 256
BC = 256
NB = N // BR

_SC_CORES = 2
_SC_SUBCORES = 16
_SC_LANES = 16
_NW = _SC_CORES * _SC_SUBCORES
_BPW = N // _NW
_CH = 128
_NCH = _BPW // _CH


def _nn_body(bounds_ref, xr_ref, xfull_ref, batch_ref, out_ref):
    i = pl.program_id(0)
    xr = xr_ref[...]
    sq_r = jnp.sum(xr * xr, axis=1)
    rb = batch_ref[0, pl.ds(i * BR, BR)]
    row_ids = i * BR + lax.broadcasted_iota(jnp.int32, (BR, BC), 0)
    col_iota = lax.broadcasted_iota(jnp.int32, (BR, BC), 1)

    cb_lo = bounds_ref[0, i]
    cb_hi = bounds_ref[1, i]

    def body(cb, carry):
        best, bidx = carry
        xc = xfull_ref[pl.ds(cb * BC, BC), :]
        sq_c = jnp.sum(xc * xc, axis=1)
        prod = lax.dot_general(xr, xc, (((1,), (1,)), ((), ())),
                               preferred_element_type=jnp.float32)
        dist = sq_r[:, None] + sq_c[None, :] - 2.0 * prod
        col_b = batch_ref[0, pl.ds(cb * BC, BC)]
        invalid = (rb[:, None] != col_b[None, :]) | (row_ids == cb * BC + col_iota)
        dist = jnp.where(invalid, jnp.inf, dist)
        m = jnp.min(dist, axis=1)
        amin = jnp.min(jnp.where(dist == m[:, None], col_iota, BC), axis=1) + cb * BC
        better = m < best
        return jnp.where(better, m, best), jnp.where(better, amin, bidx)

    best0 = jnp.full((BR,), jnp.inf, jnp.float32)
    bidx0 = i * BR + lax.broadcasted_iota(jnp.int32, (BR, 1), 0)[:, 0]
    _, bidx = lax.fori_loop(cb_lo, cb_hi, body, (best0, bidx0))
    out_ref[0, 0, :] = bidx


def _nn_idx(x, batch32, bounds):
    batch2d = batch32.reshape(1, N)
    out = pl.pallas_call(
        _nn_body,
        grid=(NB,),
        in_specs=[
            pl.BlockSpec(memory_space=pltpu.SMEM),
            pl.BlockSpec((BR, D), lambda i: (i, 0)),
            pl.BlockSpec((N, D), lambda i: (0, 0)),
            pl.BlockSpec((1, N), lambda i: (0, 0)),
        ],
        out_specs=pl.BlockSpec((1, 1, BR), lambda i: (i, 0, 0)),
        out_shape=jax.ShapeDtypeStruct((NB, 1, BR), jnp.int32),
    )(bounds, x, x, batch2d)
    return out.reshape(N)


def _sc_diff_body(x_hbm, idx_hbm, out_hbm, idx_v, own_v, nb_v, sem):
    wid = lax.axis_index("s") * _SC_CORES + lax.axis_index("c")
    base = wid * _BPW
    pltpu.sync_copy(idx_hbm.at[pl.ds(wid * _NCH, _NCH)], idx_v)
    pltpu.sync_copy(x_hbm.at[pl.ds(base, _BPW)], own_v)
    copies = [
        pltpu.async_copy(x_hbm.at[idx_v.at[k]], nb_v.at[pl.ds(k * _CH, _CH)], sem)
        for k in range(_NCH)
    ]
    for c in copies:
        c.wait()

    def row_body(r, _):
        for d0 in range(0, D, _SC_LANES):
            sl = pl.ds(d0, _SC_LANES)
            own_v[r, sl] = own_v[r, sl] - nb_v[r, sl]
        return 0

    lax.fori_loop(0, _BPW, row_body, 0)
    pltpu.sync_copy(own_v, out_hbm.at[pl.ds(base, _BPW)])


@functools.cache
def _sc_diff():
    return pl.kernel(
        _sc_diff_body,
        out_type=jax.ShapeDtypeStruct((N, D), jnp.float32),
        mesh=plsc.VectorSubcoreMesh(core_axis_name="c", subcore_axis_name="s"),
        scratch_types=[
            pltpu.VMEM((_NCH, _CH), jnp.int32),
            pltpu.VMEM((_BPW, D), jnp.float32),
            pltpu.VMEM((_BPW, D), jnp.float32),
            pltpu.SemaphoreType.DMA,
        ],
    )


def kernel(x, batch):
    batch32 = batch.astype(jnp.int32)
    b2 = batch32.reshape(NB, BR)
    col_lo = jnp.searchsorted(batch32, b2[:, 0], side="left").astype(jnp.int32)
    col_hi = jnp.searchsorted(batch32, b2[:, -1], side="right").astype(jnp.int32)
    bounds = jnp.stack([col_lo // BC, (col_hi + BC - 1) // BC])
    nn_idx = _nn_idx(x, batch32, bounds)
    return _sc_diff()(x, nn_idx.reshape(N // _CH, _CH))


# speedup vs baseline: 2.0628x; 1.9063x over previous
import functools

import jax
import jax.numpy as jnp
from jax import lax
from jax.experimental import pallas as pl
from jax.experimental.pallas import tpu as pltpu
from jax.experimental.pallas import tpu_sc as plsc

N = 8192
D = 128
BR = 256
BC = 256
NB = N // BR

_SC_CORES = 2
_SC_SUBCORES = 16
_SC_LANES = 16
_NW = _SC_CORES * _SC_SUBCORES
_BPW = N // _NW
_CH = 128
_NCH = _BPW // _CH


def _sq_body(x_ref, out_ref):
    x = x_ref[...]
    out_ref[...] = jnp.sum(x * x, axis=1, keepdims=True)


def _row_sq(x):
    return pl.pallas_call(
        _sq_body,
        out_shape=jax.ShapeDtypeStruct((N, 1), jnp.float32),
    )(x)


def _nn_body(bounds_ref, xr_ref, rb_ref, xf_ref, sqc_ref, cb_ref,
             out_ref, best_ref, blk_ref):
    i = pl.program_id(0)
    xr0 = xr_ref[...]
    sq_r = jnp.sum(xr0 * xr0, axis=1, keepdims=True)
    xr = xr0 * -2.0
    rb = rb_ref[...]
    col_iota = lax.broadcasted_iota(jnp.int32, (BR, BC), 1).astype(jnp.float32)
    eye = lax.broadcasted_iota(jnp.int32, (BR, BC), 0) == \
        lax.broadcasted_iota(jnp.int32, (BR, BC), 1)

    cb_lo = bounds_ref[0, i]
    cb_hi = bounds_ref[1, i]

    def masked_dist(cb):
        xct = xf_ref[:, pl.ds(cb * BC, BC)]
        prod = lax.dot_general(xr, xct, (((1,), (0,)), ((), ())),
                               preferred_element_type=jnp.float32)
        sq_c = sqc_ref[:, pl.ds(cb * BC, BC)]
        dist = (sq_r + sq_c) + prod
        col_b = cb_ref[:, pl.ds(cb * BC, BC)]
        invalid = (rb != col_b) | (eye & (cb == i))
        return jnp.where(invalid, jnp.inf, dist)

    best_ref[...] = masked_dist(cb_lo)
    blk_ref[...] = jnp.full((BR, BC), 1.0, jnp.float32) * cb_lo.astype(
        jnp.float32)

    def process(cb):
        dist = masked_dist(cb)
        improved = dist < best_ref[...]
        best_ref[...] = jnp.where(improved, dist, best_ref[...])
        blk_ref[...] = jnp.where(improved, cb.astype(jnp.float32),
                                 blk_ref[...])

    def body(t, _):
        a = cb_lo + 1 + 2 * t
        process(jnp.minimum(a, N // BC - 1))
        process(jnp.minimum(a + 1, N // BC - 1))
        return 0

    lax.fori_loop(0, (cb_hi - cb_lo) // 2, body, 0)

    bestv = best_ref[...]
    m = jnp.min(bestv, axis=1, keepdims=True)
    gidx = blk_ref[...] * float(BC) + col_iota
    amin = jnp.min(jnp.where(bestv == m, gidx, jnp.float32(1e9)),
                   axis=1, keepdims=True)
    self_idx = (i * BR + lax.broadcasted_iota(jnp.int32, (BR, 1), 0)
                ).astype(jnp.float32)
    bidx = jnp.where(jnp.isfinite(m), amin, self_idx)
    out_ref[...] = bidx.astype(jnp.int32)


def _nn_idx(x, batch32, bounds, sq_row):
    out = pl.pallas_call(
        _nn_body,
        grid=(NB,),
        in_specs=[
            pl.BlockSpec(memory_space=pltpu.SMEM),
            pl.BlockSpec((BR, D), lambda i: (i, 0)),
            pl.BlockSpec((BR, 1), lambda i: (i, 0)),
            pl.BlockSpec((D, N), lambda i: (0, 0)),
            pl.BlockSpec((1, N), lambda i: (0, 0)),
            pl.BlockSpec((1, N), lambda i: (0, 0)),
        ],
        out_specs=pl.BlockSpec((BR, 1), lambda i: (i, 0)),
        out_shape=jax.ShapeDtypeStruct((N, 1), jnp.int32),
        scratch_shapes=[pltpu.VMEM((BR, BC), jnp.float32),
                        pltpu.VMEM((BR, BC), jnp.float32)],
    )(bounds, x, batch32.reshape(N, 1), x.T, sq_row,
      batch32.reshape(1, N))
    return out.reshape(N)


def _sc_diff_body(x_hbm, idx_hbm, out_hbm, idx_v, own_v, nb_v, sem):
    wid = lax.axis_index("s") * _SC_CORES + lax.axis_index("c")
    base = wid * _BPW
    pltpu.sync_copy(idx_hbm.at[pl.ds(wid * _NCH, _NCH)], idx_v)
    pltpu.sync_copy(x_hbm.at[pl.ds(base, _BPW)], own_v)
    copies = [
        pltpu.async_copy(x_hbm.at[idx_v.at[k]], nb_v.at[pl.ds(k * _CH, _CH)], sem)
        for k in range(_NCH)
    ]
    for c in copies:
        c.wait()

    def row_body(r4, _):
        for rr in range(4):
            r = r4 * 4 + rr
            for d0 in range(0, D, _SC_LANES):
                sl = pl.ds(d0, _SC_LANES)
                own_v[r, sl] = own_v[r, sl] - nb_v[r, sl]
        return 0

    lax.fori_loop(0, _BPW // 4, row_body, 0)
    pltpu.sync_copy(own_v, out_hbm.at[pl.ds(base, _BPW)])


@functools.cache
def _sc_diff():
    return pl.kernel(
        _sc_diff_body,
        out_type=jax.ShapeDtypeStruct((N, D), jnp.float32),
        mesh=plsc.VectorSubcoreMesh(core_axis_name="c", subcore_axis_name="s"),
        scratch_types=[
            pltpu.VMEM((_NCH, _CH), jnp.int32),
            pltpu.VMEM((_BPW, D), jnp.float32),
            pltpu.VMEM((_BPW, D), jnp.float32),
            pltpu.SemaphoreType.DMA,
        ],
    )


def kernel(x, batch):
    batch32 = batch.astype(jnp.int32)
    b2 = batch32.reshape(NB, BR)
    sq_row = _row_sq(x).reshape(1, N)
    segs = jnp.arange(1, 9, dtype=jnp.int32)
    seg_bound = jnp.sum((batch32[:, None] < segs[None, :]), axis=0,
                        dtype=jnp.int32)
    seg_start = jnp.concatenate(
        [jnp.zeros((1,), jnp.int32), seg_bound[:7]])
    col_lo = jnp.take(seg_start, b2[:, 0])
    col_hi = jnp.take(seg_bound, b2[:, -1])
    bounds = jnp.stack([col_lo // BC, (col_hi + BC - 1) // BC])
    nn_idx = _nn_idx(x, batch32, bounds, sq_row)
    return _sc_diff()(x, nn_idx.reshape(N // _CH, _CH))
